# baseline (speedup 1.0000x reference)
import functools

import jax
import jax.numpy as jnp
from jax import lax
from jax.experimental import pallas as pl
from jax.experimental.pallas import tpu as pltpu
from jax.experimental.pallas import tpu_sc as plsc

CLS = 1000
FEAT = 64
N = 16384
L = 16
NC = 2
NS = 16
NW = NC * NS
BPW = N // NW
HPW = N // NS
HROW = HPW // 128
MROW = BPW // 128
HIST_PAD = 1024


def _body(x_hbm, lab_hbm, centre_hbm, out_hbm,
          x_v, centre_v, labm_v, labh_v, ones_v, hist_v,
          inv_v, mmst_v, acc_v, hist_sh, mm_sh,
          sem_x, sem_c, sem_l):
    c = lax.axis_index("c")
    s = lax.axis_index("s")
    wid = c * NS + s

    pltpu.sync_copy(lab_hbm.at[pl.ds(s * HROW, HROW)], labh_v)
    pending = [
        pltpu.async_copy(
            x_hbm.at[pl.ds(0, FEAT), pl.ds(wid * BPW, BPW)], x_v, sem_x),
        pltpu.async_copy(centre_hbm, centre_v, sem_c),
        pltpu.async_copy(lab_hbm.at[pl.ds(wid * MROW, MROW)], labm_v, sem_l),
    ]

    zero16 = jnp.zeros((L,), jnp.float32)

    def zstep(j, t):
        hist_v[pl.ds(j * L, L)] = zero16
        return t

    lax.fori_loop(0, HIST_PAD // L, zstep, 0)

    @pl.when(s == 0)
    def _():
        pltpu.sync_copy(hist_v, hist_sh)

    def mmstep(r, mm):
        mn, mx = mm
        for k in range(128 // L):
            v = labh_v[r, pl.ds(k * L, L)].astype(jnp.float32)
            mn = jnp.minimum(mn, v)
            mx = jnp.maximum(mx, v)
        return mn, mx

    minv, maxv = lax.fori_loop(
        0, HROW, mmstep,
        (jnp.full((L,), 1e9, jnp.float32), jnp.full((L,), -1e9, jnp.float32)))
    mmst_v[pl.ds(0, L)] = minv
    mmst_v[pl.ds(L, L)] = maxv
    pltpu.sync_copy(mmst_v, mm_sh.at[pl.ds(2 * L * s, 2 * L)])
    plsc.subcore_barrier()

    pltpu.sync_copy(mm_sh, inv_v)

    def gstep(r, mm):
        mn, mx = mm
        mn = jnp.minimum(mn, inv_v[pl.ds(2 * L * r, L)])
        mx = jnp.maximum(mx, inv_v[pl.ds(2 * L * r + L, L)])
        return mn, mx

    minv, maxv = lax.fori_loop(0, NS, gstep, (minv, maxv))
    vmin, vmax = minv[0], maxv[0]
    for j in range(1, L):
        vmin = jnp.minimum(vmin, minv[j])
        vmax = jnp.maximum(vmax, maxv[j])
    span = vmax - vmin
    span = jnp.where(span == 0.0, 1.0, span)

    ones16 = jnp.ones((L,), jnp.float32)
    for k in range(128 // L):
        ones_v[pl.ds(k * L, L)] = ones16
    def bstep(r, t0):
        for k in range(128 // L):
            lab = labh_v[r, pl.ds(k * L, L)].astype(jnp.float32)
            t = (lab - vmin) / span * jnp.float32(CLS)
            b = jnp.clip(t.astype(jnp.int32), 0, CLS - 1)
            labh_v[r, pl.ds(k * L, L)] = b
        return t0

    lax.fori_loop(0, HROW, bstep, 0)
    for r in range(HROW):
        pltpu.sync_copy(ones_v, hist_sh.at[labh_v.at[r]], add=True)
    plsc.subcore_barrier()

    pltpu.sync_copy(hist_sh, hist_v)
    labm_done = pending.pop()
    labm_done.wait()
    def cstep(r, t0):
        for k in range(128 // L):
            lab = labm_v[r, pl.ds(k * L, L)]
            cnt = plsc.load_gather(hist_v, [lab])
            inv_v[pl.ds(r * 128 + k * L, L)] = 1.0 / cnt
        return t0

    lax.fori_loop(0, MROW, cstep, 0)

    for h in pending:
        h.wait()

    def step(blk, acc):
        labc = labm_v[blk // (128 // L), pl.ds((blk % (128 // L)) * L, L)]
        invc = inv_v[pl.ds(blk * L, L)]
        col = blk * L
        fvec = jnp.zeros((L,), jnp.int32)
        v = jnp.zeros((L,), jnp.float32)
        for f in range(FEAT):
            xv = x_v[f, pl.ds(col, L)]
            cv = plsc.load_gather(centre_v, [fvec, labc])
            d = xv - cv
            v = v + d * d
            if f + 1 < FEAT:
                fvec = fvec + 1
        return acc + v * invc

    acc = lax.fori_loop(0, BPW // L, step, jnp.zeros((L,), jnp.float32))
    acc_v[...] = acc
    pltpu.sync_copy(acc_v, out_hbm.at[pl.ds(wid * L, L)])


_sc_call = functools.partial(
    pl.kernel,
    mesh=plsc.VectorSubcoreMesh(core_axis_name="c", subcore_axis_name="s"),
    out_type=jax.ShapeDtypeStruct((NW * L,), jnp.float32),
    scratch_types=[
        pltpu.VMEM((FEAT, BPW), jnp.float32),
        pltpu.VMEM((FEAT, CLS), jnp.float32),
        pltpu.VMEM((MROW, 128), jnp.int32),
        pltpu.VMEM((HROW, 128), jnp.int32),
        pltpu.VMEM((128,), jnp.float32),
        pltpu.VMEM((HIST_PAD,), jnp.float32),
        pltpu.VMEM((BPW,), jnp.float32),
        pltpu.VMEM((2 * L,), jnp.float32),
        pltpu.VMEM((L,), jnp.float32),
        pltpu.VMEM_SHARED((HIST_PAD,), jnp.float32),
        pltpu.VMEM_SHARED((2 * L * NS,), jnp.float32),
        pltpu.SemaphoreType.DMA,
        pltpu.SemaphoreType.DMA,
        pltpu.SemaphoreType.DMA,
    ],
    compiler_params=pltpu.CompilerParams(needs_layout_passes=False),
)(_body)


def kernel(x, labels, centre):
    partial = _sc_call(x.T, labels.reshape(N // 128, 128), centre.T)
    return jnp.sqrt(jnp.sum(partial))


# speedup vs baseline: 1.1911x; 1.1911x over previous
import functools

import jax
import jax.numpy as jnp
from jax import lax
from jax.experimental import pallas as pl
from jax.experimental.pallas import tpu as pltpu
from jax.experimental.pallas import tpu_sc as plsc

CLS = 1000
FEAT = 64
N = 16384
L = 16
NC = 2
NS = 16
NW = NC * NS
BPW = N // NW
HPW = N // NS
HROW = HPW // 128
MROW = BPW // 128
FGRP = 4
EGRP = NW // FGRP
FPT = FEAT // FGRP
EPT = N // EGRP
MROW2 = EPT // 128
HIST_PAD = 1024


def _body(x_hbm, lab_hbm, centre_hbm, out_hbm,
          x_v, centre_v, labm_v, labh_v, ones_v, hist_v,
          inv_v, mmst_v, acc_v, hist_sh, mm_sh,
          sem_x, sem_c, sem_l):
    c = lax.axis_index("c")
    s = lax.axis_index("s")
    wid = c * NS + s
    fgrp = wid // EGRP
    egrp = wid % EGRP

    pltpu.sync_copy(lab_hbm.at[pl.ds(s * HROW, HROW)], labh_v)
    pending = [
        pltpu.async_copy(
            x_hbm.at[pl.ds(fgrp * FPT, FPT), pl.ds(egrp * EPT, EPT)],
            x_v, sem_x),
        pltpu.async_copy(centre_hbm.at[pl.ds(fgrp * FPT, FPT)], centre_v, sem_c),
        pltpu.async_copy(lab_hbm.at[pl.ds(egrp * MROW2, MROW2)], labm_v, sem_l),
    ]

    zero16 = jnp.zeros((L,), jnp.float32)

    def zstep(j, t):
        hist_v[pl.ds(j * L, L)] = zero16
        return t

    lax.fori_loop(0, HIST_PAD // L, zstep, 0)

    @pl.when(s == 0)
    def _():
        pltpu.sync_copy(hist_v, hist_sh)

    def mmstep(r, mm):
        mn, mx = mm
        for k in range(128 // L):
            v = labh_v[r, pl.ds(k * L, L)].astype(jnp.float32)
            mn = jnp.minimum(mn, v)
            mx = jnp.maximum(mx, v)
        return mn, mx

    minv, maxv = lax.fori_loop(
        0, HROW, mmstep,
        (jnp.full((L,), 1e9, jnp.float32), jnp.full((L,), -1e9, jnp.float32)))
    mmst_v[pl.ds(0, L)] = minv
    mmst_v[pl.ds(L, L)] = maxv
    pltpu.sync_copy(mmst_v, mm_sh.at[pl.ds(2 * L * s, 2 * L)])
    plsc.subcore_barrier()

    pltpu.sync_copy(mm_sh, inv_v.at[pl.ds(0, 2 * L * NS)])

    def gstep(r, mm):
        mn, mx = mm
        mn = jnp.minimum(mn, inv_v[pl.ds(2 * L * r, L)])
        mx = jnp.maximum(mx, inv_v[pl.ds(2 * L * r + L, L)])
        return mn, mx

    minv, maxv = lax.fori_loop(0, NS, gstep, (minv, maxv))
    vmin, vmax = minv[0], maxv[0]
    for j in range(1, L):
        vmin = jnp.minimum(vmin, minv[j])
        vmax = jnp.maximum(vmax, maxv[j])
    span = vmax - vmin
    span = jnp.where(span == 0.0, 1.0, span)

    ones16 = jnp.ones((L,), jnp.float32)
    for k in range(128 // L):
        ones_v[pl.ds(k * L, L)] = ones16
    def bstep(r, t0):
        for k in range(128 // L):
            lab = labh_v[r, pl.ds(k * L, L)].astype(jnp.float32)
            t = (lab - vmin) / span * jnp.float32(CLS)
            b = jnp.clip(t.astype(jnp.int32), 0, CLS - 1)
            labh_v[r, pl.ds(k * L, L)] = b
        return t0

    lax.fori_loop(0, HROW, bstep, 0)
    for r in range(HROW):
        pltpu.sync_copy(ones_v, hist_sh.at[labh_v.at[r]], add=True)
    plsc.subcore_barrier()

    pltpu.sync_copy(hist_sh, hist_v)
    labm_done = pending.pop()
    labm_done.wait()
    def cstep(r, t0):
        for k in range(128 // L):
            lab = labm_v[r, pl.ds(k * L, L)]
            cnt = plsc.load_gather(hist_v, [lab])
            inv_v[pl.ds(r * 128 + k * L, L)] = 1.0 / cnt
        return t0

    lax.fori_loop(0, MROW2, cstep, 0)

    for h in pending:
        h.wait()

    def step(blk, acc):
        labc = labm_v[blk // (128 // L), pl.ds((blk % (128 // L)) * L, L)]
        invc = inv_v[pl.ds(blk * L, L)]
        col = blk * L
        fvec = jnp.zeros((L,), jnp.int32)
        v = jnp.zeros((L,), jnp.float32)
        for f in range(FPT):
            xv = x_v[f, pl.ds(col, L)]
            cv = plsc.load_gather(centre_v, [fvec, labc])
            d = xv - cv
            v = v + d * d
            if f + 1 < FPT:
                fvec = fvec + 1
        return acc + v * invc

    acc = lax.fori_loop(0, EPT // L, step, jnp.zeros((L,), jnp.float32))
    acc_v[...] = acc
    pltpu.sync_copy(acc_v, out_hbm.at[pl.ds(wid * L, L)])


_sc_call = functools.partial(
    pl.kernel,
    mesh=plsc.VectorSubcoreMesh(core_axis_name="c", subcore_axis_name="s"),
    out_type=jax.ShapeDtypeStruct((NW * L,), jnp.float32),
    scratch_types=[
        pltpu.VMEM((FPT, EPT), jnp.float32),
        pltpu.VMEM((FPT, CLS), jnp.float32),
        pltpu.VMEM((MROW2, 128), jnp.int32),
        pltpu.VMEM((HROW, 128), jnp.int32),
        pltpu.VMEM((128,), jnp.float32),
        pltpu.VMEM((HIST_PAD,), jnp.float32),
        pltpu.VMEM((EPT,), jnp.float32),
        pltpu.VMEM((2 * L,), jnp.float32),
        pltpu.VMEM((L,), jnp.float32),
        pltpu.VMEM_SHARED((HIST_PAD,), jnp.float32),
        pltpu.VMEM_SHARED((2 * L * NS,), jnp.float32),
        pltpu.SemaphoreType.DMA,
        pltpu.SemaphoreType.DMA,
        pltpu.SemaphoreType.DMA,
    ],
    compiler_params=pltpu.CompilerParams(needs_layout_passes=False),
)(_body)


def kernel(x, labels, centre):
    partial = _sc_call(x.T, labels.reshape(N // 128, 128), centre.T)
    return jnp.sqrt(jnp.sum(partial))
